# Initial kernel scaffold; baseline (speedup 1.0000x reference)
import jax
import jax.numpy as jnp
from jax.experimental import pallas as pl


def kernel(m_u, m_v, m_y, batch_u, batch_v, batch_y, t_u, t_v, t_y, W, b):
    raise NotImplementedError("write your pallas kernel here")


# speedup vs baseline: 2.8860x; 2.8860x over previous
import functools

import jax
import jax.numpy as jnp
from jax import lax
from jax.experimental import pallas as pl
from jax.experimental.pallas import tpu as pltpu
from jax.experimental.pallas import tpu_sc as plsc

N = 50000
B = 256
D = 320
NPLANES = 3
COLS = 32
SLICES = D // COLS
NTASKS = NPLANES * SLICES
CHUNK = 400
NCHUNKS = N // CHUNK

_mesh = plsc.VectorSubcoreMesh(core_axis_name="c", subcore_axis_name="s")


@functools.partial(
    pl.kernel,
    out_type=(
        jax.ShapeDtypeStruct((NPLANES, B, D), jnp.float32),
        jax.ShapeDtypeStruct((NPLANES, B, D), jnp.float32),
    ),
    mesh=_mesh,
    compiler_params=pltpu.CompilerParams(use_tc_tiling_on_sc=False),
    scratch_types=dict(
        dbuf0=pltpu.VMEM((CHUNK, COLS), jnp.float32),
        dbuf1=pltpu.VMEM((CHUNK, COLS), jnp.float32),
        bbuf0=pltpu.VMEM((CHUNK,), jnp.int32),
        bbuf1=pltpu.VMEM((CHUNK,), jnp.int32),
        tbuf=pltpu.VMEM((NPLANES, 16), jnp.float32),
        acc_s=pltpu.VMEM((B, COLS), jnp.float32),
        acc_a=pltpu.VMEM((B, COLS), jnp.float32),
        dsem0=pltpu.SemaphoreType.DMA,
        dsem1=pltpu.SemaphoreType.DMA,
        bsem0=pltpu.SemaphoreType.DMA,
        bsem1=pltpu.SemaphoreType.DMA,
        tsem=pltpu.SemaphoreType.DMA,
    ),
)
def _sc_pool(m_u, m_v, m_y, b_u, b_v, b_y, ts, s_out, a_out,
             dbuf0, dbuf1, bbuf0, bbuf1, tbuf, acc_s, acc_a,
             dsem0, dsem1, bsem0, bsem1, tsem):
    wid = lax.axis_index("s") * 2 + lax.axis_index("c")
    slice_i = wid % SLICES
    plane = wid // SLICES
    c0 = slice_i * COLS

    pltpu.async_copy(ts, tbuf, tsem).wait()

    dbufs = (dbuf0, dbuf1)
    bbufs = (bbuf0, bbuf1)
    dsems = (dsem0, dsem1)
    bsems = (bsem0, bsem1)

    def run_task(mref, bref, p):
        t_vec = tbuf[p, pl.ds(0, 16)]

        @pl.loop(0, B)
        def _zero(s):
            z = jnp.zeros((16,), jnp.float32)
            acc_s[s, pl.ds(0, 16)] = z
            acc_s[s, pl.ds(16, 16)] = z
            acc_a[s, pl.ds(0, 16)] = z
            acc_a[s, pl.ds(16, 16)] = z

        def start(chunk, par):
            r0 = chunk * CHUNK
            pltpu.make_async_copy(
                mref.at[pl.ds(r0, CHUNK), pl.ds(c0, COLS)],
                dbufs[par], dsems[par]).start()
            pltpu.make_async_copy(
                bref.at[pl.ds(r0, CHUNK)], bbufs[par], bsems[par]).start()

        def wait(chunk, par):
            r0 = chunk * CHUNK
            pltpu.make_async_copy(
                mref.at[pl.ds(r0, CHUNK), pl.ds(c0, COLS)],
                dbufs[par], dsems[par]).wait()
            pltpu.make_async_copy(
                bref.at[pl.ds(r0, CHUNK)], bbufs[par], bsems[par]).wait()

        def process(par):
            dbuf = dbufs[par]
            bbuf = bbufs[par]

            @pl.loop(0, CHUNK // 16)
            def _grp(g):
                bvec = bbuf[pl.ds(g * 16, 16)]
                for r in range(16):
                    s = bvec[r]
                    row = g * 16 + r
                    x0 = dbuf[row, pl.ds(0, 16)]
                    x1 = dbuf[row, pl.ds(16, 16)]
                    e0 = jnp.exp(x0 * t_vec)
                    e1 = jnp.exp(x1 * t_vec)
                    g0 = e0 * x0
                    g1 = e1 * x1
                    plsc.addupdate(acc_s.at[s, pl.ds(0, 16)], e0)
                    plsc.addupdate(acc_s.at[s, pl.ds(16, 16)], e1)
                    plsc.addupdate(acc_a.at[s, pl.ds(0, 16)], g0)
                    plsc.addupdate(acc_a.at[s, pl.ds(16, 16)], g1)

        start(0, 0)
        start(1, 1)

        @pl.loop(0, NCHUNKS - 1, step=2)
        def _pair(k):
            for par in (0, 1):
                c = k + par
                wait(c, par)
                process(par)

                @pl.when(c + 2 < NCHUNKS)
                def _():
                    start(c + 2, par)

        wait(NCHUNKS - 1, 0)
        process(0)

        pltpu.sync_copy(acc_s, s_out.at[p, :, pl.ds(c0, COLS)])
        pltpu.sync_copy(acc_a, a_out.at[p, :, pl.ds(c0, COLS)])

    @pl.when(wid < NTASKS)
    def _():
        for p, (mref, bref) in enumerate(
                ((m_u, b_u), (m_v, b_v), (m_y, b_y))):
            @pl.when(plane == p)
            def _():
                run_task(mref, bref, p)


def _tc_finalize(s_ref, a_ref, wt_ref, b_ref, o_ref):
    feats = []
    for p in range(NPLANES):
        sp = s_ref[p]
        ap = a_ref[p]
        feats.append(jnp.where(sp > 0.0, ap / sp, 0.0))
    f = jnp.concatenate(feats, axis=1)
    o_ref[...] = (
        jnp.dot(f, wt_ref[...], preferred_element_type=jnp.float32)
        + b_ref[...]
    )


def kernel(m_u, m_v, m_y, batch_u, batch_v, batch_y, t_u, t_v, t_y, W, b):
    xu = m_u.reshape(N, D)
    xv = m_v.reshape(N, D)
    xy = m_y.reshape(N, D)
    ts = jnp.broadcast_to(
        jnp.stack([t_u, t_v, t_y]).astype(jnp.float32)[:, None],
        (NPLANES, 16))
    s_all, a_all = _sc_pool(xu, xv, xy, batch_u, batch_v, batch_y, ts)

    out = pl.pallas_call(
        _tc_finalize,
        out_shape=jax.ShapeDtypeStruct((B, 3), jnp.float32),
    )(s_all, a_all, W.T, b.reshape(1, 3))
    return out


# speedup vs baseline: 3.0932x; 1.0718x over previous
import functools

import jax
import jax.numpy as jnp
from jax import lax
from jax.experimental import pallas as pl
from jax.experimental.pallas import tpu as pltpu
from jax.experimental.pallas import tpu_sc as plsc

N = 50000
B = 256
D = 320
DP = 384
NPLANES = 3
RSPLITS = 4
CHUNK = 80
_BASE_CH = 156
NPAD = 50176

_mesh = plsc.VectorSubcoreMesh(core_axis_name="c", subcore_axis_name="s")


@functools.partial(
    pl.kernel,
    out_type=(
        jax.ShapeDtypeStruct((NPLANES, RSPLITS, B, DP), jnp.float32),
        jax.ShapeDtypeStruct((NPLANES, RSPLITS, B, DP), jnp.float32),
    ),
    mesh=_mesh,
    scratch_types=dict(
        dbuf=pltpu.VMEM((2, CHUNK, 128), jnp.float32),
        nbuf=pltpu.VMEM((2, CHUNK, 64), jnp.float32),
        bbuf=pltpu.VMEM((2, 128), jnp.int32),
        tbuf=pltpu.VMEM((NPLANES, 16), jnp.float32),
        acc_s=pltpu.VMEM((B, 128), jnp.float32),
        acc_a=pltpu.VMEM((B, 128), jnp.float32),
        dsem=pltpu.SemaphoreType.DMA,
        bsem=pltpu.SemaphoreType.DMA,
        tsem=pltpu.SemaphoreType.DMA,
    ),
)
def _sc_pool(m_u, m_v, m_y, b_u, b_v, b_y, ts, s_out, a_out,
             dbuf, nbuf, bbuf, tbuf, acc_s, acc_a, dsem, bsem, tsem):
    wid = lax.axis_index("s") * 2 + lax.axis_index("c")
    pltpu.async_copy(ts, tbuf, tsem).wait()

    def run_task(mref, bref, p, c0, r, width):
        ch0 = _BASE_CH * r
        nch = jnp.where(r == RSPLITS - 1, _BASE_CH + 1, _BASE_CH)
        t_vec = tbuf[p, pl.ds(0, 16)]
        nj = width // 16
        xbuf = dbuf if width == 128 else nbuf

        @pl.loop(0, B)
        def _zero(s):
            z = jnp.zeros((16,), jnp.float32)
            for j in range(8):
                acc_s[s, pl.ds(16 * j, 16)] = z
                acc_a[s, pl.ds(16 * j, 16)] = z

        def start(k, par):
            r0 = pl.multiple_of((ch0 + k) * CHUNK, 8)
            pltpu.make_async_copy(
                mref.at[pl.ds(r0, CHUNK), pl.ds(c0, width)],
                xbuf.at[par], dsem).start()
            pltpu.make_async_copy(
                bref.at[pl.ds(r0, 128)], bbuf.at[par], bsem).start()

        def wait(k, par):
            r0 = pl.multiple_of((ch0 + k) * CHUNK, 8)
            pltpu.make_async_copy(
                mref.at[pl.ds(r0, CHUNK), pl.ds(c0, width)],
                xbuf.at[par], dsem).wait()
            pltpu.make_async_copy(
                bref.at[pl.ds(r0, 128)], bbuf.at[par], bsem).wait()

        def process(par):
            @pl.loop(0, CHUNK // 16)
            def _grp(g):
                base = g * 16
                bvec = bbuf[par, pl.ds(base, 16)]
                for rr in range(16):
                    s = bvec[rr]
                    row = base + rr
                    for j in range(nj):
                        x = xbuf[par, row, pl.ds(16 * j, 16)]
                        e = jnp.exp(x * t_vec)
                        gg = e * x
                        plsc.addupdate(acc_s.at[s, pl.ds(16 * j, 16)], e)
                        plsc.addupdate(acc_a.at[s, pl.ds(16 * j, 16)], gg)

        start(0, 0)
        start(1, 1)

        @pl.loop(0, nch)
        def _chunk(k):
            par = lax.rem(k, 2)
            wait(k, par)
            process(par)

            @pl.when(k + 2 < nch)
            def _():
                start(k + 2, par)

        pltpu.sync_copy(acc_s, s_out.at[p, r, :, pl.ds(c0, 128)])
        pltpu.sync_copy(acc_a, a_out.at[p, r, :, pl.ds(c0, 128)])

    for p, (mref, bref) in enumerate(((m_u, b_u), (m_v, b_v), (m_y, b_y))):
        @pl.when((wid < 24) & (wid // 8 == p))
        def _():
            g = (wid % 8) // 4
            c0 = pl.multiple_of(g * 128, 128)
            run_task(mref, bref, p, c0, wid % 4, 128)

    @pl.when((wid >= 24) & (wid < 28))
    def _():
        run_task(m_u, b_u, 0, 256, wid - 24, 64)

    @pl.when(wid >= 28)
    def _():
        run_task(m_v, b_v, 1, 256, wid - 28, 64)

    @pl.when((wid >= 24) & (wid < 28))
    def _():
        run_task(m_y, b_y, 2, 256, wid - 24, 64)


def _tc_finalize(sp_ref, ap_ref, wt_ref, b_ref, o_ref):
    feats = []
    for p in range(NPLANES):
        sp = sp_ref[p, 0]
        ap = ap_ref[p, 0]
        for rr in range(1, RSPLITS):
            sp = sp + sp_ref[p, rr]
            ap = ap + ap_ref[p, rr]
        feats.append(jnp.where(sp > 0.0, ap / sp, 0.0)[:, :D])
    f = jnp.concatenate(feats, axis=1)
    o_ref[...] = (
        jnp.dot(f, wt_ref[...], preferred_element_type=jnp.float32)
        + b_ref[...]
    )


def kernel(m_u, m_v, m_y, batch_u, batch_v, batch_y, t_u, t_v, t_y, W, b):
    xu = m_u.reshape(N, D)
    xv = m_v.reshape(N, D)
    xy = m_y.reshape(N, D)
    pad = NPAD - N
    bu = jnp.pad(batch_u, (0, pad))
    bv = jnp.pad(batch_v, (0, pad))
    by = jnp.pad(batch_y, (0, pad))
    ts = jnp.broadcast_to(
        jnp.stack([t_u, t_v, t_y]).astype(jnp.float32)[:, None],
        (NPLANES, 16))
    s_all, a_all = _sc_pool(xu, xv, xy, bu, bv, by, ts)

    out = pl.pallas_call(
        _tc_finalize,
        out_shape=jax.ShapeDtypeStruct((B, 3), jnp.float32),
    )(s_all, a_all, W.T, b.reshape(1, 3))
    return out


# speedup vs baseline: 7.8098x; 2.5248x over previous
import functools

import jax
import jax.numpy as jnp
from jax import lax
from jax.experimental import pallas as pl
from jax.experimental.pallas import tpu as pltpu
from jax.experimental.pallas import tpu_sc as plsc

N = 50000
B = 256
D = 320
DP = 384
NPLANES = 3
RSPLITS = 4
CHUNK = 80
_BASE_CH = 156
NPAD = 50176

_mesh = plsc.VectorSubcoreMesh(core_axis_name="c", subcore_axis_name="s")


@functools.partial(
    pl.kernel,
    out_type=(
        jax.ShapeDtypeStruct((NPLANES, RSPLITS, B, DP), jnp.float32),
        jax.ShapeDtypeStruct((NPLANES, RSPLITS, B, DP), jnp.float32),
    ),
    mesh=_mesh,
    scratch_types=dict(
        dbuf=pltpu.VMEM((2, CHUNK, 128), jnp.float32),
        nbuf=pltpu.VMEM((2, CHUNK, 64), jnp.float32),
        bbuf=pltpu.VMEM((2, 128), jnp.int32),
        tbuf=pltpu.VMEM((NPLANES, 16), jnp.float32),
        acc_s=pltpu.VMEM((B, 128), jnp.float32),
        acc_a=pltpu.VMEM((B, 128), jnp.float32),
        dsem=pltpu.SemaphoreType.DMA,
        bsem=pltpu.SemaphoreType.DMA,
        tsem=pltpu.SemaphoreType.DMA,
    ),
)
def _sc_pool(m_u, m_v, m_y, b_u, b_v, b_y, ts, s_out, a_out,
             dbuf, nbuf, bbuf, tbuf, acc_s, acc_a, dsem, bsem, tsem):
    wid = lax.axis_index("s") * 2 + lax.axis_index("c")
    pltpu.async_copy(ts, tbuf, tsem).wait()

    def run_task(mref, bref, p, c0, r, width):
        ch0 = _BASE_CH * r
        nch = jnp.where(r == RSPLITS - 1, _BASE_CH + 1, _BASE_CH)
        t_vec = tbuf[p, pl.ds(0, 16)]
        nj = width // 16
        xbuf = dbuf if width == 128 else nbuf

        @pl.loop(0, B)
        def _zero(s):
            z = jnp.zeros((16,), jnp.float32)
            for j in range(8):
                acc_s[s, pl.ds(16 * j, 16)] = z
                acc_a[s, pl.ds(16 * j, 16)] = z

        def start(k, par):
            r0 = pl.multiple_of((ch0 + k) * CHUNK, 8)
            pltpu.make_async_copy(
                mref.at[pl.ds(r0, CHUNK), pl.ds(c0, width)],
                xbuf.at[par], dsem).start()
            pltpu.make_async_copy(
                bref.at[pl.ds(r0, 128)], bbuf.at[par], bsem).start()

        def wait(k, par):
            r0 = pl.multiple_of((ch0 + k) * CHUNK, 8)
            pltpu.make_async_copy(
                mref.at[pl.ds(r0, CHUNK), pl.ds(c0, width)],
                xbuf.at[par], dsem).wait()
            pltpu.make_async_copy(
                bref.at[pl.ds(r0, 128)], bbuf.at[par], bsem).wait()

        def process(par):
            @pl.loop(0, CHUNK // 16)
            def _grp(g):
                base = g * 16
                bvec = bbuf[par, pl.ds(base, 16)]
                s0 = bvec[0]
                s15 = bvec[15]

                @pl.when(s0 == s15)
                def _uniform():
                    for j in range(nj):
                        ep = []
                        gp = []
                        for q in range(4):
                            es = []
                            gs = []
                            for rr in range(4):
                                row = base + 4 * q + rr
                                x = xbuf[par, row, pl.ds(16 * j, 16)]
                                e = jnp.exp(x * t_vec)
                                es.append(e)
                                gs.append(e * x)
                            ep.append((es[0] + es[1]) + (es[2] + es[3]))
                            gp.append((gs[0] + gs[1]) + (gs[2] + gs[3]))
                        esum = (ep[0] + ep[1]) + (ep[2] + ep[3])
                        gsum = (gp[0] + gp[1]) + (gp[2] + gp[3])
                        plsc.addupdate(acc_s.at[s0, pl.ds(16 * j, 16)], esum)
                        plsc.addupdate(acc_a.at[s0, pl.ds(16 * j, 16)], gsum)

                @pl.when(s0 != s15)
                def _mixed():
                    for rr in range(16):
                        s = bvec[rr]
                        row = base + rr
                        for j in range(nj):
                            x = xbuf[par, row, pl.ds(16 * j, 16)]
                            e = jnp.exp(x * t_vec)
                            gg = e * x
                            plsc.addupdate(acc_s.at[s, pl.ds(16 * j, 16)], e)
                            plsc.addupdate(acc_a.at[s, pl.ds(16 * j, 16)], gg)

        start(0, 0)
        start(1, 1)

        @pl.loop(0, nch)
        def _chunk(k):
            par = lax.rem(k, 2)
            wait(k, par)
            process(par)

            @pl.when(k + 2 < nch)
            def _():
                start(k + 2, par)

        pltpu.sync_copy(acc_s, s_out.at[p, r, :, pl.ds(c0, 128)])
        pltpu.sync_copy(acc_a, a_out.at[p, r, :, pl.ds(c0, 128)])

    for p, (mref, bref) in enumerate(((m_u, b_u), (m_v, b_v), (m_y, b_y))):
        @pl.when((wid < 24) & (wid // 8 == p))
        def _():
            g = (wid % 8) // 4
            c0 = pl.multiple_of(g * 128, 128)
            run_task(mref, bref, p, c0, wid % 4, 128)

    @pl.when((wid >= 24) & (wid < 28))
    def _():
        run_task(m_u, b_u, 0, 256, wid - 24, 64)

    @pl.when(wid >= 28)
    def _():
        run_task(m_v, b_v, 1, 256, wid - 28, 64)

    @pl.when((wid >= 24) & (wid < 28))
    def _():
        run_task(m_y, b_y, 2, 256, wid - 24, 64)


def _tc_finalize(sp_ref, ap_ref, wt_ref, b_ref, o_ref):
    feats = []
    for p in range(NPLANES):
        sp = sp_ref[p, 0]
        ap = ap_ref[p, 0]
        for rr in range(1, RSPLITS):
            sp = sp + sp_ref[p, rr]
            ap = ap + ap_ref[p, rr]
        feats.append(jnp.where(sp > 0.0, ap / sp, 0.0)[:, :D])
    f = jnp.concatenate(feats, axis=1)
    o_ref[...] = (
        jnp.dot(f, wt_ref[...], preferred_element_type=jnp.float32)
        + b_ref[...]
    )


def kernel(m_u, m_v, m_y, batch_u, batch_v, batch_y, t_u, t_v, t_y, W, b):
    xu = m_u.reshape(N, D)
    xv = m_v.reshape(N, D)
    xy = m_y.reshape(N, D)
    pad = NPAD - N
    bu = jnp.pad(batch_u, (0, pad))
    bv = jnp.pad(batch_v, (0, pad))
    by = jnp.pad(batch_y, (0, pad))
    ts = jnp.broadcast_to(
        jnp.stack([t_u, t_v, t_y]).astype(jnp.float32)[:, None],
        (NPLANES, 16))
    s_all, a_all = _sc_pool(xu, xv, xy, bu, bv, by, ts)

    out = pl.pallas_call(
        _tc_finalize,
        out_shape=jax.ShapeDtypeStruct((B, 3), jnp.float32),
    )(s_all, a_all, W.T, b.reshape(1, 3))
    return out
